# baseline (speedup 1.0000x reference)
import functools
import math

import jax
import jax.numpy as jnp
from jax.experimental import pallas as pl
from jax.experimental.pallas import tpu as pltpu

BLOCK = 128
NEG_INF = -1e30


def _mask_body(q_ref, k_ref, mask_ref, *, nb, block, scale, min_retain, max_retain):
    d = q_ref.shape[-1]
    qh = q_ref[0]
    kh = k_ref[0]
    qp = qh.reshape(nb, block, d).mean(axis=1)
    kp = kh.reshape(nb, block, d).mean(axis=1)
    s = jax.lax.dot_general(qp, kp, (((1,), (1,)), ((), ())),
                            preferred_element_type=jnp.float32) * scale
    m = jnp.max(s, axis=-1, keepdims=True)
    e = jnp.exp(s - m)
    p = e / jnp.sum(e, axis=-1, keepdims=True)

    col_ids = jax.lax.broadcasted_iota(jnp.int32, (nb, nb), 1)
    rank = jnp.zeros((nb, nb), jnp.float32)
    for j in range(nb):
        col = p[:, j:j + 1]
        gt = jnp.sum((p > col).astype(jnp.float32), axis=-1, keepdims=True)
        if j > 0:
            eq = jnp.sum((p[:, :j] == col).astype(jnp.float32), axis=-1,
                         keepdims=True)
        else:
            eq = jnp.zeros_like(gt)
        rank = rank + (gt + eq) * (col_ids == j).astype(jnp.float32)

    cum = jnp.zeros((nb, nb), jnp.float32)
    for i in range(nb):
        le = (rank <= float(i)).astype(jnp.float32)
        ci = jnp.sum(p * le, axis=-1, keepdims=True)
        cum = cum + ci * (col_ids == i).astype(jnp.float32)

    thr = 0.95 * cum[:, nb - 1:nb]
    kcnt = jnp.sum((cum < thr).astype(jnp.float32), axis=-1, keepdims=True)
    kk = jnp.clip(kcnt, float(min_retain), float(max_retain))
    mask_ref[0] = (rank < kk).astype(jnp.int32)


def _attn_body(mask_smem, q_ref, k_ref, v_ref, out_ref, s_scr, acc_scr,
               *, nb, block, scale):
    h = pl.program_id(0)
    i = pl.program_id(1)
    base = (h * nb + i) * nb

    qb = q_ref[0]
    s_scr[...] = jnp.full(s_scr.shape, NEG_INF, jnp.float32)
    for j in range(nb):
        @pl.when(mask_smem[base + j] == 1)
        def _(j=j):
            kj = k_ref[0, pl.ds(j * block, block), :]
            s_scr[:, pl.ds(j * block, block)] = jax.lax.dot_general(
                qb, kj, (((1,), (1,)), ((), ())),
                preferred_element_type=jnp.float32) * scale

    s = s_scr[...]
    m = jnp.max(s, axis=-1, keepdims=True)
    p = jnp.exp(s - m)
    p = p / jnp.sum(p, axis=-1, keepdims=True)

    acc_scr[...] = jnp.zeros(acc_scr.shape, jnp.float32)
    for j in range(nb):
        @pl.when(mask_smem[base + j] == 1)
        def _(j=j):
            vj = v_ref[0, pl.ds(j * block, block), :]
            pj = p[:, j * block:(j + 1) * block]
            acc_scr[...] += jax.lax.dot_general(
                pj, vj, (((1,), (0,)), ((), ())),
                preferred_element_type=jnp.float32)
    out_ref[0] = acc_scr[...]


@jax.jit
def kernel(q, k, v):
    B, H, S, d = q.shape
    nb = S // BLOCK
    BH = B * H
    scale = 1.0 / math.sqrt(d)
    min_retain = max(1, int(nb * 0.05))
    max_retain = max(1, int(nb * 0.7))

    qf = q.reshape(BH, S, d)
    kf = k.reshape(BH, S, d)
    vf = v.reshape(BH, S, d)

    mask = pl.pallas_call(
        functools.partial(_mask_body, nb=nb, block=BLOCK, scale=scale,
                          min_retain=min_retain, max_retain=max_retain),
        grid=(BH,),
        in_specs=[
            pl.BlockSpec((1, S, d), lambda h: (h, 0, 0)),
            pl.BlockSpec((1, S, d), lambda h: (h, 0, 0)),
        ],
        out_specs=pl.BlockSpec((1, nb, nb), lambda h: (h, 0, 0)),
        out_shape=jax.ShapeDtypeStruct((BH, nb, nb), jnp.int32),
        compiler_params=pltpu.CompilerParams(
            dimension_semantics=("arbitrary",)),
    )(qf, kf)

    mask_flat = mask.reshape(-1)

    grid_spec = pltpu.PrefetchScalarGridSpec(
        num_scalar_prefetch=1,
        grid=(BH, nb),
        in_specs=[
            pl.BlockSpec((1, BLOCK, d), lambda h, i, m: (h, i, 0)),
            pl.BlockSpec((1, S, d), lambda h, i, m: (h, 0, 0)),
            pl.BlockSpec((1, S, d), lambda h, i, m: (h, 0, 0)),
        ],
        out_specs=pl.BlockSpec((1, BLOCK, d), lambda h, i, m: (h, i, 0)),
        scratch_shapes=[
            pltpu.VMEM((BLOCK, S), jnp.float32),
            pltpu.VMEM((BLOCK, d), jnp.float32),
        ],
    )
    out = pl.pallas_call(
        functools.partial(_attn_body, nb=nb, block=BLOCK, scale=scale),
        grid_spec=grid_spec,
        out_shape=jax.ShapeDtypeStruct((BH, S, d), jnp.float32),
        compiler_params=pltpu.CompilerParams(
            dimension_semantics=("parallel", "arbitrary")),
    )(mask_flat, qf, kf, vf)

    return out.reshape(B, H, S, d)
